# Initial kernel scaffold; baseline (speedup 1.0000x reference)
import jax
import jax.numpy as jnp
from jax.experimental import pallas as pl
from jax.experimental.pallas import tpu as pltpu
from jax.experimental.pallas import tpu_sc as plsc

_K = 10
_POOL = 100000
_TILE = 2048
_NT = 49
_POOL_PAD = _TILE * _NT
_Q = 1024
_D = 128
_SLOTS = 32
_BIGI = jnp.int32(2**30)


def _topk_body(x_ref, p_ref, oi_ref, rd_ref, ri_ref):
    t = pl.program_id(0)
    inf = jnp.float32(jnp.inf)

    @pl.when(t == 0)
    def _init():
        rd_ref[...] = jnp.full((_Q, _SLOTS), inf, jnp.float32)
        ri_ref[...] = jnp.full((_Q, _SLOTS), _BIGI, jnp.int32)

    x = x_ref[...]
    tile = p_ref[...]
    xx = jnp.sum(x * x, axis=1, keepdims=True)
    ones = jnp.ones((8, _D), jnp.float32)
    pp8 = jax.lax.dot_general(ones, tile * tile, (((1,), (1,)), ((), ())),
                              preferred_element_type=jnp.float32)
    pp = pp8[0:1, :]
    xt = jax.lax.dot_general(x, tile, (((1,), (1,)), ((), ())),
                             preferred_element_type=jnp.float32)
    d2 = xx + pp - 2.0 * xt
    dist = jnp.sqrt(jnp.maximum(d2, 0.0))
    colid = t * _TILE + jax.lax.broadcasted_iota(jnp.int32, (_Q, _TILE), 1)
    dist = jnp.where(colid >= _POOL, inf, dist)

    all_d = jnp.concatenate([rd_ref[...], dist], axis=1)
    all_i = jnp.concatenate([ri_ref[...], colid], axis=1)

    new_d, new_i = [], []
    for _ in range(_K):
        m = jnp.min(all_d, axis=1, keepdims=True)
        im = jnp.min(jnp.where(all_d == m, all_i, _BIGI), axis=1,
                     keepdims=True)
        new_d.append(m)
        new_i.append(im)
        all_d = jnp.where(all_i == im, inf, all_d)

    pad_d = jnp.full((_Q, _SLOTS - _K), inf, jnp.float32)
    pad_i = jnp.full((_Q, _SLOTS - _K), _BIGI, jnp.int32)
    rd_ref[...] = jnp.concatenate(new_d + [pad_d], axis=1)
    ri_ref[...] = jnp.concatenate(new_i + [pad_i], axis=1)
    oi_ref[...] = ri_ref[...]


def _topk_indices(x, pool_pad):
    return pl.pallas_call(
        _topk_body,
        grid=(_NT,),
        in_specs=[pl.BlockSpec((_Q, _D), lambda t: (0, 0)),
                  pl.BlockSpec((_TILE, _D), lambda t: (t, 0))],
        out_specs=pl.BlockSpec((_Q, _SLOTS), lambda t: (0, 0)),
        out_shape=jax.ShapeDtypeStruct((_Q, _SLOTS), jnp.int32),
        scratch_shapes=[pltpu.VMEM((_Q, _SLOTS), jnp.float32),
                        pltpu.VMEM((_Q, _SLOTS), jnp.int32)],
    )(x, pool_pad)


def _sc_gather(pool_x, pool_y, idx_flat):
    n = idx_flat.shape[0]
    w = 128
    mesh = plsc.VectorSubcoreMesh(core_axis_name="core",
                                  subcore_axis_name="subcore")
    idx2 = idx_flat.reshape(1, n)

    @pl.kernel(out_type=(jax.ShapeDtypeStruct((n, _D), jnp.float32),
                         jax.ShapeDtypeStruct((n, 1), jnp.float32)),
               mesh=mesh)
    def gk(px_hbm, py_hbm, i_hbm, ox_hbm, oy_hbm):
        def body(i_vmem, ox_vmem, oy_vmem):
            pltpu.sync_copy(px_hbm.at[i_vmem.at[0]], ox_vmem)
            pltpu.sync_copy(py_hbm.at[i_vmem.at[0]], oy_vmem)

        pltpu.emit_pipeline(
            body,
            grid=(n // w,),
            in_specs=[pl.BlockSpec((1, w), lambda i: (0, i))],
            out_specs=[pl.BlockSpec((w, _D), lambda i: (i, 0)),
                       pl.BlockSpec((w, 1), lambda i: (i, 0))],
            core_axis_name=("core", "subcore"),
            dimension_semantics=(pltpu.PARALLEL,),
        )(i_hbm, ox_hbm, oy_hbm)

    return gk(pool_x, pool_y, idx2)


def kernel(x, pool_x, pool_y):
    pool_pad = jnp.pad(pool_x, ((0, _POOL_PAD - _POOL), (0, 0)))
    out_idx = _topk_indices(x, pool_pad)
    idx = out_idx[:, :_K].reshape(-1)
    gx, gy = _sc_gather(pool_x, pool_y, idx)
    return jnp.concatenate([gx.reshape(_Q, _K, _D), gy.reshape(_Q, _K, 1)],
                           axis=-1)


# speedup vs baseline: 45.6782x; 45.6782x over previous
import jax
import jax.numpy as jnp
from jax.experimental import pallas as pl
from jax.experimental.pallas import tpu as pltpu
from jax.experimental.pallas import tpu_sc as plsc

_K = 10
_POOL = 100000
_TILE = 2048
_NT = 49
_POOL_PAD = _TILE * _NT
_Q = 1024
_D = 128
_SLOTS = 128
_BIGI = 2**30


def _topk_body(x_ref, xx_ref, p_ref, pp_ref, oi_ref, rd_ref, ri_ref):
    t = pl.program_id(0)
    inf = float('inf')

    @pl.when(t == 0)
    def _init():
        rd_ref[...] = jnp.full((_Q, _SLOTS), inf, jnp.float32)
        ri_ref[...] = jnp.full((_Q, _SLOTS), _BIGI, jnp.int32)

    x = x_ref[...]
    tile = p_ref[...]
    xx = xx_ref[...]
    pp = pp_ref[0]
    xt = jax.lax.dot_general(x.astype(jnp.bfloat16), tile.astype(jnp.bfloat16),
                             (((1,), (1,)), ((), ())),
                             preferred_element_type=jnp.float32)
    d2 = xx + pp - 2.0 * xt
    dist = jnp.sqrt(jnp.maximum(d2, 0.0))
    colid = t * _TILE + jax.lax.broadcasted_iota(jnp.int32, (_Q, _TILE), 1)

    tile_d, tile_i = [], []
    for _ in range(_K):
        m = jnp.min(dist, axis=1, keepdims=True)
        im = jnp.min(jnp.where(dist == m, colid, _BIGI), axis=1,
                     keepdims=True)
        tile_d.append(m)
        tile_i.append(im)
        dist = jnp.where(colid == im, inf, dist)

    pad_d = jnp.full((_Q, _SLOTS - _K), inf, jnp.float32)
    pad_i = jnp.full((_Q, _SLOTS - _K), _BIGI, jnp.int32)
    cand_d = jnp.concatenate(tile_d + [pad_d], axis=1)
    cand_i = jnp.concatenate(tile_i + [pad_i], axis=1)

    all_d = jnp.concatenate([rd_ref[...], cand_d], axis=1)
    all_i = jnp.concatenate([ri_ref[...], cand_i], axis=1)
    new_d, new_i = [], []
    for _ in range(_K):
        m = jnp.min(all_d, axis=1, keepdims=True)
        im = jnp.min(jnp.where(all_d == m, all_i, _BIGI), axis=1,
                     keepdims=True)
        new_d.append(m)
        new_i.append(im)
        all_d = jnp.where(all_i == im, inf, all_d)

    rd_ref[...] = jnp.concatenate(new_d + [pad_d], axis=1)
    ri_ref[...] = jnp.concatenate(new_i + [pad_i], axis=1)
    oi_ref[...] = ri_ref[...]


def _topk_indices(x, xx, pool_pad, pp_pad):
    return pl.pallas_call(
        _topk_body,
        grid=(_NT,),
        in_specs=[pl.BlockSpec((_Q, _D), lambda t: (0, 0)),
                  pl.BlockSpec((_Q, 1), lambda t: (0, 0)),
                  pl.BlockSpec((_TILE, _D), lambda t: (t, 0)),
                  pl.BlockSpec((1, 1, _TILE), lambda t: (t, 0, 0))],
        out_specs=pl.BlockSpec((_Q, _SLOTS), lambda t: (0, 0)),
        out_shape=jax.ShapeDtypeStruct((_Q, _SLOTS), jnp.int32),
        scratch_shapes=[pltpu.VMEM((_Q, _SLOTS), jnp.float32),
                        pltpu.VMEM((_Q, _SLOTS), jnp.int32)],
    )(x, xx, pool_pad, pp_pad)


def _sc_gather(pool_x, pool_y2d, idx_flat, yrow_idx):
    n = idx_flat.shape[0]
    w = 128
    mesh = plsc.VectorSubcoreMesh(core_axis_name="core",
                                  subcore_axis_name="subcore")
    idx2 = idx_flat.reshape(1, n)
    yrow2 = yrow_idx.reshape(1, n)

    @pl.kernel(out_type=(jax.ShapeDtypeStruct((n, _D), jnp.float32),
                         jax.ShapeDtypeStruct((n, _D), jnp.float32)),
               mesh=mesh)
    def gk(px_hbm, py_hbm, i_hbm, iy_hbm, ox_hbm, oy_hbm):
        def body(i_vmem, iy_vmem, ox_vmem, oy_vmem):
            pltpu.sync_copy(px_hbm.at[i_vmem.at[0]], ox_vmem)
            pltpu.sync_copy(py_hbm.at[iy_vmem.at[0]], oy_vmem)

        pltpu.emit_pipeline(
            body,
            grid=(n // w,),
            in_specs=[pl.BlockSpec((1, w), lambda i: (0, i)),
                      pl.BlockSpec((1, w), lambda i: (0, i))],
            out_specs=[pl.BlockSpec((w, _D), lambda i: (i, 0)),
                       pl.BlockSpec((w, _D), lambda i: (i, 0))],
            core_axis_name=("core", "subcore"),
            dimension_semantics=(pltpu.PARALLEL,),
        )(i_hbm, iy_hbm, ox_hbm, oy_hbm)

    return gk(pool_x, pool_y2d, idx2, yrow2)


def _ysel_body(yrow_ref, lane_ref, oy_ref):
    lanes = jax.lax.broadcasted_iota(jnp.int32, (_Q * _K, _D), 1)
    sel = jnp.where(lanes == lane_ref[...], yrow_ref[...], 0.0)
    oy_ref[...] = jnp.sum(sel, axis=1, keepdims=True)


def _y_select(gyrow, lane):
    return pl.pallas_call(
        _ysel_body,
        in_specs=[pl.BlockSpec((_Q * _K, _D), lambda: (0, 0)),
                  pl.BlockSpec((_Q * _K, 1), lambda: (0, 0))],
        out_specs=pl.BlockSpec((_Q * _K, 1), lambda: (0, 0)),
        out_shape=jax.ShapeDtypeStruct((_Q * _K, 1), jnp.float32),
    )(gyrow, lane)


def kernel(x, pool_x, pool_y):
    pool_pad = jnp.pad(pool_x, ((0, _POOL_PAD - _POOL), (0, 0)))
    pool_y2d = jnp.pad(pool_y.reshape(-1),
                       (0, _POOL_PAD - _POOL)).reshape(_POOL_PAD // _D, _D)
    xx = jnp.sum(x * x, axis=-1, keepdims=True)
    pp = jnp.sum(pool_x * pool_x, axis=-1)
    pp_pad = jnp.pad(pp, (0, _POOL_PAD - _POOL),
                     constant_values=jnp.inf).reshape(_NT, 1, _TILE)
    out_idx = _topk_indices(x, xx, pool_pad, pp_pad)
    idx = out_idx[:, :_K].reshape(-1)
    gx, gyrow = _sc_gather(pool_x, pool_y2d, idx, idx // _D)
    gy = _y_select(gyrow, (idx % _D).reshape(-1, 1))
    return jnp.concatenate([gx.reshape(_Q, _K, _D), gy.reshape(_Q, _K, 1)],
                           axis=-1)


# speedup vs baseline: 107.5823x; 2.3552x over previous
import jax
import jax.numpy as jnp
from jax.experimental import pallas as pl
from jax.experimental.pallas import tpu as pltpu
from jax.experimental.pallas import tpu_sc as plsc

_K = 10
_POOL = 100000
_TILE = 2048
_NT = 49
_POOL_PAD = _TILE * _NT
_Q = 1024
_D = 128
_CHUNKS = _TILE // _D
_NG = _NT * _CHUNKS
_NSEL = 16
_BIGI = 2**30


def _dist_body(x_ref, xx_ref, p_ref, pp_ref, d2_ref, gm_ref):
    x = x_ref[...]
    tile = p_ref[...]
    xx = xx_ref[...]
    pp = pp_ref[0]
    xt = jax.lax.dot_general(x.astype(jnp.bfloat16), tile.astype(jnp.bfloat16),
                             (((1,), (1,)), ((), ())),
                             preferred_element_type=jnp.float32)
    d2 = xx + pp - 2.0 * xt
    d2_ref[...] = d2
    cms = [jnp.min(d2[:, c * _D:(c + 1) * _D], axis=1, keepdims=True)
           for c in range(_CHUNKS)]
    gm_ref[0] = jnp.concatenate(cms, axis=1)


def _dist_and_groupmin(x, xx, pool_pad, pp_pad):
    return pl.pallas_call(
        _dist_body,
        grid=(_NT,),
        in_specs=[pl.BlockSpec((_Q, _D), lambda t: (0, 0)),
                  pl.BlockSpec((_Q, 1), lambda t: (0, 0)),
                  pl.BlockSpec((_TILE, _D), lambda t: (t, 0)),
                  pl.BlockSpec((1, 1, _TILE), lambda t: (t, 0, 0))],
        out_specs=[pl.BlockSpec((_Q, _TILE), lambda t: (0, t)),
                   pl.BlockSpec((1, _Q, _CHUNKS), lambda t: (t, 0, 0))],
        out_shape=[jax.ShapeDtypeStruct((_Q, _POOL_PAD), jnp.float32),
                   jax.ShapeDtypeStruct((_NT, _Q, _CHUNKS), jnp.float32)],
    )(x, xx, pool_pad, pp_pad)


def _gext_body(gm_ref, og_ref):
    gm = gm_ref[...]
    inf = float('inf')
    gid = jax.lax.broadcasted_iota(jnp.int32, (_Q, _NG), 1)
    sel = []
    for _ in range(_NSEL):
        m = jnp.min(gm, axis=1, keepdims=True)
        im = jnp.min(jnp.where(gm == m, gid, _BIGI), axis=1, keepdims=True)
        sel.append(im)
        gm = jnp.where(gid == im, inf, gm)
    og_ref[...] = jnp.concatenate(sel, axis=1)


def _group_extract(gm2):
    return pl.pallas_call(
        _gext_body,
        in_specs=[pl.BlockSpec((_Q, _NG), lambda: (0, 0))],
        out_specs=pl.BlockSpec((_Q, _NSEL), lambda: (0, 0)),
        out_shape=jax.ShapeDtypeStruct((_Q, _NSEL), jnp.int32),
    )(gm2)


def _fin_body(d2c_ref, col_ref, oi_ref):
    inf = float('inf')
    dist = jnp.sqrt(jnp.maximum(d2c_ref[...], 0.0))
    colid = col_ref[...]
    new_i = []
    for _ in range(_K):
        m = jnp.min(dist, axis=1, keepdims=True)
        im = jnp.min(jnp.where(dist == m, colid, _BIGI), axis=1, keepdims=True)
        new_i.append(im)
        dist = jnp.where(colid == im, inf, dist)
    pad_i = jnp.full((_Q, 16 - _K), _BIGI, jnp.int32)
    oi_ref[...] = jnp.concatenate(new_i + [pad_i], axis=1)


def _final_topk(d2cand, colcand):
    w = _NSEL * _D
    return pl.pallas_call(
        _fin_body,
        in_specs=[pl.BlockSpec((_Q, w), lambda: (0, 0)),
                  pl.BlockSpec((_Q, w), lambda: (0, 0))],
        out_specs=pl.BlockSpec((_Q, 16), lambda: (0, 0)),
        out_shape=jax.ShapeDtypeStruct((_Q, 16), jnp.int32),
    )(d2cand, colcand)


def _sc_mesh():
    return plsc.VectorSubcoreMesh(core_axis_name="core",
                                  subcore_axis_name="subcore")


def _sc_gather1(src, idx_flat):
    n = idx_flat.shape[0]
    w = 128

    @pl.kernel(out_type=jax.ShapeDtypeStruct((n, _D), src.dtype),
               mesh=_sc_mesh())
    def gk(src_hbm, i_hbm, o_hbm):
        def body(i_vmem, o_vmem):
            pltpu.sync_copy(src_hbm.at[i_vmem.at[0]], o_vmem)

        pltpu.emit_pipeline(
            body,
            grid=(n // w,),
            in_specs=[pl.BlockSpec((1, w), lambda i: (0, i))],
            out_specs=[pl.BlockSpec((w, _D), lambda i: (i, 0))],
            core_axis_name=("core", "subcore"),
            dimension_semantics=(pltpu.PARALLEL,),
        )(i_hbm, o_hbm)

    return gk(src, idx_flat.reshape(1, n))


def _sc_gather(pool_x, pool_y2d, idx_flat, yrow_idx):
    n = idx_flat.shape[0]
    w = 128

    @pl.kernel(out_type=(jax.ShapeDtypeStruct((n, _D), jnp.float32),
                         jax.ShapeDtypeStruct((n, _D), jnp.float32)),
               mesh=_sc_mesh())
    def gk(px_hbm, py_hbm, i_hbm, iy_hbm, ox_hbm, oy_hbm):
        def body(i_vmem, iy_vmem, ox_vmem, oy_vmem):
            pltpu.sync_copy(px_hbm.at[i_vmem.at[0]], ox_vmem)
            pltpu.sync_copy(py_hbm.at[iy_vmem.at[0]], oy_vmem)

        pltpu.emit_pipeline(
            body,
            grid=(n // w,),
            in_specs=[pl.BlockSpec((1, w), lambda i: (0, i)),
                      pl.BlockSpec((1, w), lambda i: (0, i))],
            out_specs=[pl.BlockSpec((w, _D), lambda i: (i, 0)),
                       pl.BlockSpec((w, _D), lambda i: (i, 0))],
            core_axis_name=("core", "subcore"),
            dimension_semantics=(pltpu.PARALLEL,),
        )(i_hbm, iy_hbm, ox_hbm, oy_hbm)

    return gk(pool_x, pool_y2d, idx_flat.reshape(1, n),
              yrow_idx.reshape(1, n))


def _ysel_body(yrow_ref, lane_ref, oy_ref):
    lanes = jax.lax.broadcasted_iota(jnp.int32, (_Q * _K, _D), 1)
    sel = jnp.where(lanes == lane_ref[...], yrow_ref[...], 0.0)
    oy_ref[...] = jnp.sum(sel, axis=1, keepdims=True)


def _y_select(gyrow, lane):
    return pl.pallas_call(
        _ysel_body,
        in_specs=[pl.BlockSpec((_Q * _K, _D), lambda: (0, 0)),
                  pl.BlockSpec((_Q * _K, 1), lambda: (0, 0))],
        out_specs=pl.BlockSpec((_Q * _K, 1), lambda: (0, 0)),
        out_shape=jax.ShapeDtypeStruct((_Q * _K, 1), jnp.float32),
    )(gyrow, lane)


def kernel(x, pool_x, pool_y):
    pool_pad = jnp.pad(pool_x, ((0, _POOL_PAD - _POOL), (0, 0)))
    pool_y2d = jnp.pad(pool_y.reshape(-1),
                       (0, _POOL_PAD - _POOL)).reshape(_POOL_PAD // _D, _D)
    xx = jnp.sum(x * x, axis=-1, keepdims=True)
    pp = jnp.sum(pool_x * pool_x, axis=-1)
    pp_pad = jnp.pad(pp, (0, _POOL_PAD - _POOL),
                     constant_values=jnp.inf).reshape(_NT, 1, _TILE)

    d2_spill, gm = _dist_and_groupmin(x, xx, pool_pad, pp_pad)
    gm2 = gm.transpose(1, 0, 2).reshape(_Q, _NG)
    gids = _group_extract(gm2)

    rowidx = (jnp.arange(_Q, dtype=jnp.int32)[:, None] * _NG
              + gids).reshape(-1)
    d2cand = _sc_gather1(d2_spill.reshape(_Q * _NG, _D), rowidx)
    colcand = (gids[:, :, None] * _D
               + jnp.arange(_D, dtype=jnp.int32)).reshape(_Q, _NSEL * _D)

    fin = _final_topk(d2cand.reshape(_Q, _NSEL * _D), colcand)
    idx = fin[:, :_K].reshape(-1)

    gx, gyrow = _sc_gather(pool_x, pool_y2d, idx, idx // _D)
    gy = _y_select(gyrow, (idx % _D).reshape(-1, 1))
    return jnp.concatenate([gx.reshape(_Q, _K, _D), gy.reshape(_Q, _K, 1)],
                           axis=-1)
